# baseline (speedup 1.0000x reference)
import functools

import jax
import jax.numpy as jnp
from jax import lax
from jax.experimental import pallas as pl
from jax.experimental.pallas import tpu as pltpu
from jax.experimental.pallas import tpu_sc as plsc

N = 10000
D = 128
DW = 144
DWQ = 160
NROWS = 10016
E = 320000
EPAD = 327680
JB = 128
EROWS = EPAD // JB
SENT = N
NWORK = 32
JPW = EROWS // NWORK
NB = 4
CH = 8
TPT = NROWS // 16

SCALE1 = 256.0
SCALE2 = 1024.0

BLK = 2504


def _mm_body(x_ref, w_ref, o_ref):
    o_ref[...] = jnp.dot(x_ref[...], w_ref[...],
                         preferred_element_type=jnp.float32)


def _mm(xp, W):
    return pl.pallas_call(
        _mm_body,
        grid=(NROWS // BLK,),
        in_specs=[
            pl.BlockSpec((BLK, D), lambda i: (i, 0)),
            pl.BlockSpec((D, D), lambda i: (0, 0)),
        ],
        out_specs=pl.BlockSpec((BLK, D), lambda i: (i, 0)),
        out_shape=jax.ShapeDtypeStruct((NROWS, D), jnp.float32),
    )(xp, W)


def _mid_body(agg_ref, t_ref, b_ref, w_ref, h1_ref, t2_ref):
    s = (agg_ref[0].astype(jnp.float32) +
         agg_ref[1].astype(jnp.float32)) * (1.0 / SCALE1)
    deg = s[:, 128:129]
    h = (s[:, :128] + t_ref[...]) / (deg + 1.0) + b_ref[...]
    h1 = jnp.maximum(h, 0.0)
    h1_ref[...] = h1
    t2_ref[...] = jnp.dot(h1, w_ref[...], preferred_element_type=jnp.float32)


def _mid(agg, t1, b1, W2):
    return pl.pallas_call(
        _mid_body,
        grid=(NROWS // BLK,),
        in_specs=[
            pl.BlockSpec((2, BLK, DWQ), lambda i: (0, i, 0)),
            pl.BlockSpec((BLK, D), lambda i: (i, 0)),
            pl.BlockSpec((1, D), lambda i: (0, 0)),
            pl.BlockSpec((D, D), lambda i: (0, 0)),
        ],
        out_specs=[
            pl.BlockSpec((BLK, D), lambda i: (i, 0)),
            pl.BlockSpec((BLK, D), lambda i: (i, 0)),
        ],
        out_shape=[
            jax.ShapeDtypeStruct((NROWS, D), jnp.float32),
            jax.ShapeDtypeStruct((NROWS, D), jnp.float32),
        ],
    )(agg, t1, b1, W2)


def _fin_body(agg_ref, t_ref, b_ref, h2_ref):
    s = (agg_ref[0].astype(jnp.float32) +
         agg_ref[1].astype(jnp.float32)) * (1.0 / SCALE2)
    deg = s[:, 128:129]
    h2_ref[...] = (s[:, :128] + t_ref[...]) / (deg + 1.0) + b_ref[...]


def _fin(agg, t2, b2):
    return pl.pallas_call(
        _fin_body,
        grid=(NROWS // BLK,),
        in_specs=[
            pl.BlockSpec((2, BLK, DWQ), lambda i: (0, i, 0)),
            pl.BlockSpec((BLK, D), lambda i: (i, 0)),
            pl.BlockSpec((1, D), lambda i: (0, 0)),
        ],
        out_specs=pl.BlockSpec((BLK, D), lambda i: (i, 0)),
        out_shape=jax.ShapeDtypeStruct((NROWS, D), jnp.float32),
    )(agg, t2, b2)


def _sc_agg_body(t_hbm, src_hbm, dst_hbm, out_hbm, src_v, dst_v, rows_v,
                 acc_sh, *sems):
    gs = sems[:NB]
    ss = sems[NB:]
    cid = lax.axis_index("c")
    sid = lax.axis_index("s")
    wid = cid * 16 + sid

    z32 = jnp.zeros((32,), jnp.int16)

    def zrow(r, carry):
        for c in range(DWQ // 32):
            rows_v[0, r, 32 * c:32 * (c + 1)] = z32
        return carry

    lax.fori_loop(0, JB, zrow, 0)

    def zacc(k, carry):
        pltpu.sync_copy(rows_v.at[0],
                        acc_sh.at[pl.ds(sid * TPT + JB * k, JB)])
        return carry

    lax.fori_loop(0, TPT // JB, zacc, 0)
    _rem = TPT % JB
    pltpu.sync_copy(rows_v.at[0, pl.ds(0, _rem)],
                    acc_sh.at[pl.ds(sid * TPT + TPT - _rem, _rem)])

    wbase = wid * JPW
    pltpu.sync_copy(src_hbm.at[pl.ds(wbase, CH)], src_v.at[0])
    pltpu.sync_copy(dst_hbm.at[pl.ds(wbase, CH)], dst_v.at[0])
    plsc.subcore_barrier()

    g_desc = [
        pltpu.async_copy(t_hbm.at[src_v.at[0, b]], rows_v.at[b], gs[b])
        for b in range(NB)
    ]
    s_desc = [None] * NB
    for j in range(JPW):
        b = j % NB
        cc = j // CH
        if j % CH == 0 and j + CH < JPW:
            pltpu.sync_copy(src_hbm.at[pl.ds(wbase + j + CH, CH)],
                            src_v.at[(cc + 1) % 2])
            pltpu.sync_copy(dst_hbm.at[pl.ds(wbase + j + CH, CH)],
                            dst_v.at[(cc + 1) % 2])
        g_desc[b].wait()
        s_desc[b] = pltpu.async_copy(
            rows_v.at[b], acc_sh.at[dst_v.at[cc % 2, j % CH]], ss[b],
            add=True)
        if j + NB < JPW:
            jn = j + NB
            s_desc[b].wait()
            g_desc[b] = pltpu.async_copy(
                t_hbm.at[src_v.at[(jn // CH) % 2, jn % CH]], rows_v.at[b],
                gs[b])
    for j in range(max(JPW - NB, 0), JPW):
        s_desc[j % NB].wait()

    plsc.subcore_barrier()
    pltpu.sync_copy(acc_sh.at[pl.ds(sid * TPT, TPT)],
                    out_hbm.at[cid, pl.ds(sid * TPT, TPT)])


@functools.partial(
    pl.kernel,
    mesh=plsc.VectorSubcoreMesh(core_axis_name="c", subcore_axis_name="s"),
    compiler_params=pltpu.CompilerParams(use_tc_tiling_on_sc=False),
    out_type=jax.ShapeDtypeStruct((2, NROWS, DWQ), jnp.int16),
    scratch_types=[
        pltpu.VMEM((2, CH, JB), jnp.int32),
        pltpu.VMEM((2, CH, JB), jnp.int32),
        pltpu.VMEM((NB, JB, DWQ), jnp.int16),
        pltpu.VMEM_SHARED((NROWS, DWQ), jnp.int16),
    ] + [pltpu.SemaphoreType.DMA] * (2 * NB),
)
def _sc_agg(t_hbm, src_hbm, dst_hbm, out_hbm, src_v, dst_v, rows_v,
            acc_sh, *sems):
    _sc_agg_body(t_hbm, src_hbm, dst_hbm, out_hbm, src_v, dst_v, rows_v,
                 acc_sh, *sems)


def kernel(x, edge_index, W1, b1, W2, b2):
    src = edge_index[0]
    dst = edge_index[1]
    pad = jnp.full((EPAD - E,), SENT, jnp.int32)
    src2d = jnp.concatenate([src, pad]).reshape(EROWS, JB)
    dst2d = jnp.concatenate([dst, pad]).reshape(EROWS, JB)
    xp = jnp.pad(x, ((0, NROWS - N), (0, 0)))
    b1r = b1.reshape(1, D)
    b2r = b2.reshape(1, D)

    def publish(t, scale):
        tq = jnp.clip(jnp.round(t * scale), -32767.0, 32767.0)
        ones = jnp.full((NROWS, 1), scale, jnp.float32)
        zer = jnp.zeros((NROWS, DWQ - D - 1), jnp.float32)
        return jnp.concatenate([tq, ones, zer], axis=1).astype(jnp.int16)

    t1 = _mm(xp, W1)
    agg1 = _sc_agg(publish(t1, SCALE1), src2d, dst2d)
    h1p, t2 = _mid(agg1, t1, b1r, W2)
    agg2 = _sc_agg(publish(t2, SCALE2), src2d, dst2d)
    h2p = _fin(agg2, t2, b2r)
    return h1p[:N], h2p[:N]


# speedup vs baseline: 1.0381x; 1.0381x over previous
import functools

import jax
import jax.numpy as jnp
import numpy as np
from jax import lax
from jax.experimental import pallas as pl
from jax.experimental.pallas import tpu as pltpu
from jax.experimental.pallas import tpu_sc as plsc

N = 10000
D = 128
DW = 144
NROWS = 10016
E = 320000
EPAD = 327680
JB = 128
EROWS = EPAD // JB
SENT = N
NWORK = 32
JPW = EROWS // NWORK
CH = 4
TPT = NROWS // 16

BLK = 2504

PERM = np.array(
    [32 * k + (j % 2) * 16 + j // 2 for k in range(4) for j in range(32)],
    dtype=np.int32,
)


def _mm_body(x_ref, w_ref, o_ref):
    o_ref[...] = jnp.dot(x_ref[...], w_ref[...],
                         preferred_element_type=jnp.float32)


def _mm(xp, W):
    return pl.pallas_call(
        _mm_body,
        grid=(NROWS // BLK,),
        in_specs=[
            pl.BlockSpec((BLK, D), lambda i: (i, 0)),
            pl.BlockSpec((D, D), lambda i: (0, 0)),
        ],
        out_specs=pl.BlockSpec((BLK, D), lambda i: (i, 0)),
        out_shape=jax.ShapeDtypeStruct((NROWS, D), jnp.float32),
    )(xp, W)


def _mid_body(agg_ref, t_ref, b_ref, w_ref, h1_ref, t2_ref):
    s = agg_ref[0] + agg_ref[1]
    deg = s[:, 128:129]
    h = (s[:, :128] + t_ref[...]) / (deg + 1.0) + b_ref[...]
    h1 = jnp.maximum(h, 0.0)
    h1_ref[...] = h1
    t2_ref[...] = jnp.dot(h1, w_ref[...], preferred_element_type=jnp.float32)


def _mid(agg, t1, b1, W2):
    return pl.pallas_call(
        _mid_body,
        grid=(NROWS // BLK,),
        in_specs=[
            pl.BlockSpec((2, BLK, DW), lambda i: (0, i, 0)),
            pl.BlockSpec((BLK, D), lambda i: (i, 0)),
            pl.BlockSpec((1, D), lambda i: (0, 0)),
            pl.BlockSpec((D, D), lambda i: (0, 0)),
        ],
        out_specs=[
            pl.BlockSpec((BLK, D), lambda i: (i, 0)),
            pl.BlockSpec((BLK, D), lambda i: (i, 0)),
        ],
        out_shape=[
            jax.ShapeDtypeStruct((NROWS, D), jnp.float32),
            jax.ShapeDtypeStruct((NROWS, D), jnp.float32),
        ],
    )(agg, t1, b1, W2)


def _fin_body(agg_ref, t_ref, b_ref, h2_ref):
    s = agg_ref[0] + agg_ref[1]
    deg = s[:, 128:129]
    h2_ref[...] = (s[:, :128] + t_ref[...]) / (deg + 1.0) + b_ref[...]


def _fin(agg, t2, b2):
    return pl.pallas_call(
        _fin_body,
        grid=(NROWS // BLK,),
        in_specs=[
            pl.BlockSpec((2, BLK, DW), lambda i: (0, i, 0)),
            pl.BlockSpec((BLK, D), lambda i: (i, 0)),
            pl.BlockSpec((1, D), lambda i: (0, 0)),
        ],
        out_specs=pl.BlockSpec((BLK, D), lambda i: (i, 0)),
        out_shape=jax.ShapeDtypeStruct((NROWS, D), jnp.float32),
    )(agg, t2, b2)


def _sc_agg_body(t_hbm, src_hbm, dst_hbm, out_hbm, src_v, dst_v, gb_v, rows_v,
                 acc_sh, *sems):
    gs = sems[:2]
    ss = sems[2]
    cid = lax.axis_index("c")
    sid = lax.axis_index("s")
    wid = cid * 16 + sid

    z16 = jnp.zeros((16,), jnp.float32)

    def zrow(r, carry):
        for c in range(DW // 16):
            rows_v[r, 16 * c:16 * (c + 1)] = z16
        return carry

    lax.fori_loop(0, JB, zrow, 0)

    def zacc(k, carry):
        pltpu.sync_copy(rows_v.at[pl.ds(0, JB)],
                        acc_sh.at[pl.ds(sid * TPT + JB * k, JB)])
        return carry

    lax.fori_loop(0, TPT // JB, zacc, 0)
    _rem = TPT % JB
    pltpu.sync_copy(rows_v.at[pl.ds(0, _rem)],
                    acc_sh.at[pl.ds(sid * TPT + TPT - _rem, _rem)])

    c16 = jnp.where(lax.iota(jnp.int32, 16) == 0, 1.0, 0.0).astype(jnp.float32)

    def tail(r, carry):
        rows_v[r, 128:144] = c16
        return carry

    lax.fori_loop(0, JB, tail, 0)

    wbase = wid * JPW
    pltpu.sync_copy(src_hbm.at[pl.ds(wbase, CH)], src_v.at[0])
    pltpu.sync_copy(dst_hbm.at[pl.ds(wbase, CH)], dst_v.at[0])
    plsc.subcore_barrier()

    g_desc = [
        pltpu.async_copy(t_hbm.at[src_v.at[0, b]], gb_v.at[b], gs[b])
        for b in range(2)
    ]
    s_desc = None
    for j in range(JPW):
        b = j % 2
        cc = j // CH
        if j % CH == 0 and j + CH < JPW:
            pltpu.sync_copy(src_hbm.at[pl.ds(wbase + j + CH, CH)],
                            src_v.at[(cc + 1) % 2])
            pltpu.sync_copy(dst_hbm.at[pl.ds(wbase + j + CH, CH)],
                            dst_v.at[(cc + 1) % 2])
        g_desc[b].wait()
        if s_desc is not None:
            s_desc.wait()

        def widen(r, carry, b=b):
            for c4 in range(4):
                v = gb_v[b, r, 16 * c4:16 * c4 + 16]
                rows_v[r, 32 * c4:32 * c4 + 16] = (
                    lax.bitcast_convert_type(
                        lax.shift_left(v, 16), jnp.float32))
                rows_v[r, 32 * c4 + 16:32 * c4 + 32] = (
                    lax.bitcast_convert_type(
                        lax.bitwise_and(v, jnp.int32(-65536)),
                        jnp.float32))
            return carry

        lax.fori_loop(0, JB, widen, 0)
        s_desc = pltpu.async_copy(
            rows_v, acc_sh.at[dst_v.at[cc % 2, j % CH]], ss, add=True)
        if j + 2 < JPW:
            jn = j + 2
            g_desc[b] = pltpu.async_copy(
                t_hbm.at[src_v.at[(jn // CH) % 2, jn % CH]], gb_v.at[b],
                gs[b])
    if s_desc is not None:
        s_desc.wait()

    plsc.subcore_barrier()
    pltpu.sync_copy(acc_sh.at[pl.ds(sid * TPT, TPT)],
                    out_hbm.at[cid, pl.ds(sid * TPT, TPT)])


@functools.partial(
    pl.kernel,
    mesh=plsc.VectorSubcoreMesh(core_axis_name="c", subcore_axis_name="s"),
    compiler_params=pltpu.CompilerParams(use_tc_tiling_on_sc=False),
    out_type=jax.ShapeDtypeStruct((2, NROWS, DW), jnp.float32),
    scratch_types=[
        pltpu.VMEM((2, CH, JB), jnp.int32),
        pltpu.VMEM((2, CH, JB), jnp.int32),
        pltpu.VMEM((2, JB, D // 2), jnp.int32),
        pltpu.VMEM((JB, DW), jnp.float32),
        pltpu.VMEM_SHARED((NROWS, DW), jnp.float32),
    ] + [pltpu.SemaphoreType.DMA] * 3,
)
def _sc_agg(t_hbm, src_hbm, dst_hbm, out_hbm, src_v, dst_v, gb_v, rows_v,
            acc_sh, *sems):
    _sc_agg_body(t_hbm, src_hbm, dst_hbm, out_hbm, src_v, dst_v, gb_v, rows_v,
                 acc_sh, *sems)


def kernel(x, edge_index, W1, b1, W2, b2):
    src = edge_index[0]
    dst = edge_index[1]
    pad = jnp.full((EPAD - E,), SENT, jnp.int32)
    src2d = jnp.concatenate([src, pad]).reshape(EROWS, JB)
    dst2d = jnp.concatenate([dst, pad]).reshape(EROWS, JB)
    xp = jnp.pad(x, ((0, NROWS - N), (0, 0)))
    b1r = b1.reshape(1, D)
    b2r = b2.reshape(1, D)
    perm = jnp.asarray(PERM)

    def publish(t):
        tb = jnp.take(t, perm, axis=1).astype(jnp.bfloat16)
        return lax.bitcast_convert_type(tb.reshape(NROWS, D // 2, 2),
                                        jnp.int32)

    t1 = _mm(xp, W1)
    agg1 = _sc_agg(publish(t1), src2d, dst2d)
    h1p, t2 = _mid(agg1, t1, b1r, W2)
    agg2 = _sc_agg(publish(t2), src2d, dst2d)
    h2p = _fin(agg2, t2, b2r)
    return h1p[:N], h2p[:N]
